# baseline (speedup 1.0000x reference)
import jax
import jax.numpy as jnp
from jax.experimental import pallas as pl

_N_LABELS = 1000
_NV = _N_LABELS - 1
_ROWS = 1024 * 50
_R = 256


def _tc_body(in_ref, out_ref):
    s = in_ref[0]
    sc = s.reshape(_R, 1) - 1
    ji = jax.lax.broadcasted_iota(jnp.int32, (_R, _NV), 1)
    out_ref[...] = (ji == sc).astype(jnp.float32)


def kernel(inputs):
    flat = inputs.reshape(_ROWS // _R, 1, _R)
    out = pl.pallas_call(
        _tc_body,
        grid=(_ROWS // _R,),
        in_specs=[pl.BlockSpec((1, 1, _R), lambda i: (i, 0, 0))],
        out_specs=pl.BlockSpec((_R, _NV), lambda i: (i, 0)),
        out_shape=jax.ShapeDtypeStruct((_ROWS, _NV), jnp.float32),
    )(flat)
    return out.reshape(1024, 50, _NV)


# speedup vs baseline: 1.5943x; 1.5943x over previous
import jax
import jax.numpy as jnp
from jax.experimental import pallas as pl

_N_LABELS = 1000
_NV = _N_LABELS - 1
_B = 32


def _tc_body(in_ref, out_ref):
    s = in_ref[...]
    ji = jax.lax.broadcasted_iota(jnp.int32, (_B, 50, _NV), 2)
    out_ref[...] = (ji == (s - 1)[:, :, None]).astype(jnp.float32)


def kernel(inputs):
    return pl.pallas_call(
        _tc_body,
        grid=(1024 // _B,),
        in_specs=[pl.BlockSpec((_B, 50), lambda i: (i, 0))],
        out_specs=pl.BlockSpec((_B, 50, _NV), lambda i: (i, 0, 0)),
        out_shape=jax.ShapeDtypeStruct((1024, 50, _NV), jnp.float32),
    )(inputs)
